# baseline (speedup 1.0000x reference)
import functools

import jax
import jax.numpy as jnp
from jax import lax
from jax.experimental import pallas as pl
from jax.experimental.pallas import tpu as pltpu
from jax.experimental.pallas import tpu_sc as plsc

_N = 10000
_E = 160000
_D = 275
_NLAYERS = 5
_NFEAT = 9
_VOCAB = 128
_B = 20
_NG = 500

_NP = 10240
_DP = 288
_HALF = 144
_NC, _NS = 2, 16
_CHUNK = 96
_RPT = _NP // _NS
_BLK = 512
_NB = _NP // _BLK

_mesh = plsc.VectorSubcoreMesh(core_axis_name="c", subcore_axis_name="s",
                               num_cores=_NC, num_subcores=_NS)


def _edge_pad(n_edges):
  unit = _NS * _CHUNK * 4
  return unit * (-(-n_edges // unit))


def _make_gather_scatter(n_edges_pad):
  ept = n_edges_pad // _NS
  cpt = ept // _CHUNK
  kq = cpt // 4

  @functools.partial(
      pl.kernel, mesh=_mesh,
      out_type=(jax.ShapeDtypeStruct((_NP, _HALF), jnp.float32),
                jax.ShapeDtypeStruct((_NP, _HALF), jnp.float32)),
      scratch_types=[
          pltpu.VMEM((4, _CHUNK), jnp.int32),
          pltpu.VMEM((4, _CHUNK), jnp.int32),
          pltpu.VMEM((_CHUNK, _HALF), jnp.float32),
          pltpu.VMEM((_CHUNK, _HALF), jnp.float32),
          pltpu.VMEM_SHARED((_NP, _HALF), jnp.float32),
          (pltpu.SemaphoreType.DMA,) * 4,
          pltpu.SemaphoreType.DMA,
          pltpu.SemaphoreType.DMA,
          pltpu.SemaphoreType.DMA,
          pltpu.SemaphoreType.DMA,
      ],
      compiler_params=pltpu.CompilerParams(use_tc_tiling_on_sc=False))
  def k(src2, dst2, tab0, tab1, init0, init1, out0, out1,
        srcb, dstb, rows0, rows1, acc, semi, semg0, semg1, sems0, sems1):
    cid = lax.axis_index("c")
    sid = lax.axis_index("s")
    r0 = sid * _RPT
    halves = ((tab0, init0, out0), (tab1, init1, out1))
    for h, (tab, init, out) in enumerate(halves):
      @pl.when(cid == h)
      def _(init=init):
        pltpu.sync_copy(init.at[pl.ds(r0, _RPT)], acc.at[pl.ds(r0, _RPT)])
    plsc.subcore_barrier()
    rowbufs = (rows0, rows1)
    semgs = (semg0, semg1)
    semss = (sems0, sems1)
    cbase = sid * cpt

    def load_idx(j, p4):
      pltpu.async_copy(src2.at[cbase + j], srcb.at[p4], semi[p4])
      pltpu.async_copy(dst2.at[cbase + j], dstb.at[p4], semi[p4])

    def wait_idx(p4):
      pltpu.make_async_copy(src2.at[0], srcb.at[p4], semi[p4]).wait()
      pltpu.make_async_copy(dst2.at[0], dstb.at[p4], semi[p4]).wait()

    for h, (tab, init, out) in enumerate(halves):
      @pl.when(cid == h)
      def _(tab=tab):
        load_idx(0, 0)
        load_idx(1, 1)
        wait_idx(0)
        pltpu.async_copy(tab.at[srcb.at[0]], rows0, semg0)
        def outer(kk, carry):
          j0 = 4 * kk
          for r in range(4):
            j = j0 + r
            p2 = r % 2
            p4 = r
            rb, sg, ss = rowbufs[p2], semgs[p2], semss[p2]
            nb, sgn, ssn = rowbufs[1 - p2], semgs[1 - p2], semss[1 - p2]
            @pl.when(j > 0)
            def _():
              pltpu.make_async_copy(nb, acc.at[dstb.at[0]], ssn).wait()
            pltpu.make_async_copy(tab.at[srcb.at[0]], rb, sg).wait()
            pltpu.async_copy(rb, acc.at[dstb.at[p4]], ss, add=True)
            @pl.when(j + 1 < cpt)
            def _():
              wait_idx((p4 + 1) % 4)
              pltpu.async_copy(tab.at[srcb.at[(p4 + 1) % 4]], nb, sgn)
            @pl.when(j + 2 < cpt)
            def _():
              load_idx(j + 2, (p4 + 2) % 4)
          return carry
        lax.fori_loop(0, kq, outer, 0)
        pltpu.make_async_copy(rows1, acc.at[dstb.at[0]], sems1).wait()
    plsc.subcore_barrier()
    for h, (tab, init, out) in enumerate(halves):
      @pl.when(cid == h)
      def _(out=out):
        pltpu.sync_copy(acc.at[pl.ds(r0, _RPT)], out.at[pl.ds(r0, _RPT)])

  return k


def _make_deg(n_edges_pad):
  epw = n_edges_pad // (_NC * _NS)
  cpw = epw // _CHUNK

  @functools.partial(
      pl.kernel, mesh=_mesh,
      out_type=(jax.ShapeDtypeStruct((_NP, 16), jnp.float32),
                jax.ShapeDtypeStruct((_NP, 16), jnp.float32)),
      scratch_types=[
          pltpu.VMEM((cpw, _CHUNK), jnp.int32),
          pltpu.VMEM((_CHUNK, 16), jnp.float32),
          pltpu.VMEM_SHARED((_NP, 16), jnp.float32),
          pltpu.SemaphoreType.DMA,
      ],
      compiler_params=pltpu.CompilerParams(use_tc_tiling_on_sc=False))
  def k(dst2, zeros16, out0, out1, dstb, rows, acc, sem):
    cid = lax.axis_index("c")
    sid = lax.axis_index("s")
    r0 = sid * _RPT
    wid = cid * _NS + sid
    pltpu.sync_copy(dst2.at[pl.ds(wid * cpw, cpw)], dstb)
    pltpu.sync_copy(zeros16.at[pl.ds(r0, _RPT)], acc.at[pl.ds(r0, _RPT)])
    one = jnp.full((16,), 1.0, jnp.float32)
    for r in range(_CHUNK):
      rows[r] = one
    plsc.subcore_barrier()
    def chunk(j, carry):
      @pl.when(j >= 2)
      def _():
        pltpu.make_async_copy(rows, acc.at[dstb.at[0]], sem).wait()
      pltpu.async_copy(rows, acc.at[dstb.at[j]], sem, add=True)
      return carry
    lax.fori_loop(0, cpw, chunk, 0)
    pltpu.make_async_copy(rows, acc.at[dstb.at[0]], sem).wait()
    pltpu.make_async_copy(rows, acc.at[dstb.at[0]], sem).wait()
    plsc.subcore_barrier()
    for h, out in enumerate((out0, out1)):
      @pl.when(cid == h)
      def _(out=out):
        pltpu.sync_copy(acc.at[pl.ds(r0, _RPT)], out.at[pl.ds(r0, _RPT)])

  return k


def _dinv_body(d0_r, d1_r, o_r):
  g = pl.program_id(0)
  deg = 1.0 + d0_r[...][:, :1] + d1_r[...][:, :1]
  row = g * _BLK + lax.broadcasted_iota(jnp.int32, (_BLK, 1), 0)
  o_r[...] = jnp.where(row < _N, lax.rsqrt(deg), 0.0)


def _dinv_call(d0, d1):
  return pl.pallas_call(
      _dinv_body,
      grid=(_NB,),
      in_specs=[pl.BlockSpec((_BLK, 16), lambda g: (g, 0)),
                pl.BlockSpec((_BLK, 16), lambda g: (g, 0))],
      out_specs=pl.BlockSpec((_BLK, 1), lambda g: (g, 0)),
      out_shape=jax.ShapeDtypeStruct((_NP, 1), jnp.float32),
  )(d0, d1)


def _mlp_body(relu, in0_r, in1_r, rsin_r, rsout_r, m_r, a_r, w_r, c4_r,
              o0_r, o1_r):
  t = jnp.concatenate([in0_r[...], in1_r[...]], axis=1)
  t = t * rsin_r[...] * m_r[...] + a_r[...]
  if relu:
    t = jnp.maximum(t, 0.0)
  y = jnp.dot(t, w_r[...], preferred_element_type=jnp.float32)
  y = y * rsout_r[...] + c4_r[...]
  o0_r[...] = y[:, :_HALF]
  o1_r[...] = y[:, _HALF:]


def _mlp_call(in0, in1, rs_in, rs_out, m, a, wt, c4, relu):
  return pl.pallas_call(
      functools.partial(_mlp_body, relu),
      grid=(_NB,),
      in_specs=[
          pl.BlockSpec((_BLK, _HALF), lambda g: (g, 0)),
          pl.BlockSpec((_BLK, _HALF), lambda g: (g, 0)),
          pl.BlockSpec((_BLK, 1), lambda g: (g, 0)),
          pl.BlockSpec((_BLK, 1), lambda g: (g, 0)),
          pl.BlockSpec((1, _DP), lambda g: (0, 0)),
          pl.BlockSpec((1, _DP), lambda g: (0, 0)),
          pl.BlockSpec((_DP, _DP), lambda g: (0, 0)),
          pl.BlockSpec((1, _DP), lambda g: (0, 0)),
      ],
      out_specs=(pl.BlockSpec((_BLK, _HALF), lambda g: (g, 0)),
                 pl.BlockSpec((_BLK, _HALF), lambda g: (g, 0))),
      out_shape=(jax.ShapeDtypeStruct((_NP, _HALF), jnp.float32),
                 jax.ShapeDtypeStruct((_NP, _HALF), jnp.float32)),
  )(in0, in1, rs_in, rs_out, m, a, wt, c4)


def _mlpbn_body(in0_r, in1_r, rs_r, rsout_r, gam_r, bet_r, w_r, c4_r,
                o0_r, o1_r, st_r):
  p = pl.program_id(0)
  g = pl.program_id(1)
  t = jnp.concatenate([in0_r[...], in1_r[...]], axis=1) * rs_r[...]
  @pl.when(p == 0)
  def _():
    @pl.when(g == 0)
    def _():
      st_r[...] = jnp.zeros_like(st_r)
    st_r[0:1, :] = st_r[0:1, :] + jnp.sum(t, axis=0, keepdims=True)
    st_r[1:2, :] = st_r[1:2, :] + jnp.sum(t * t, axis=0, keepdims=True)
  @pl.when(p == 1)
  def _():
    mu = st_r[0:1, :] * (1.0 / _N)
    var = st_r[1:2, :] * (1.0 / _N) - mu * mu
    m = gam_r[...] * lax.rsqrt(var + 1e-5)
    a = bet_r[...] - mu * m
    z = jnp.maximum(t * m + a, 0.0)
    y = jnp.dot(z, w_r[...], preferred_element_type=jnp.float32)
    y = y * rsout_r[...] + c4_r[...]
    o0_r[...] = y[:, :_HALF]
    o1_r[...] = y[:, _HALF:]


def _mlpbn_call(in0, in1, rs_in, rs_out, gam, bet, wt, c4):
  return pl.pallas_call(
      _mlpbn_body,
      grid=(2, _NB),
      in_specs=[
          pl.BlockSpec((_BLK, _HALF), lambda p, g: (g, 0)),
          pl.BlockSpec((_BLK, _HALF), lambda p, g: (g, 0)),
          pl.BlockSpec((_BLK, 1), lambda p, g: (g, 0)),
          pl.BlockSpec((_BLK, 1), lambda p, g: (g, 0)),
          pl.BlockSpec((1, _DP), lambda p, g: (0, 0)),
          pl.BlockSpec((1, _DP), lambda p, g: (0, 0)),
          pl.BlockSpec((_DP, _DP), lambda p, g: (0, 0)),
          pl.BlockSpec((1, _DP), lambda p, g: (0, 0)),
      ],
      out_specs=(pl.BlockSpec((_BLK, _HALF), lambda p, g: (g, 0)),
                 pl.BlockSpec((_BLK, _HALF), lambda p, g: (g, 0))),
      out_shape=(jax.ShapeDtypeStruct((_NP, _HALF), jnp.float32),
                 jax.ShapeDtypeStruct((_NP, _HALF), jnp.float32)),
      scratch_shapes=[pltpu.VMEM((8, _DP), jnp.float32)],
  )(in0, in1, rs_in, rs_out, gam, bet, wt, c4)


def _logits_body(h_r, o_r):
  xb = h_r[0]
  o_r[0] = lax.dot_general(xb, xb, (((1,), (1,)), ((), ())),
                           preferred_element_type=jnp.float32)


def _logits_call(hb):
  return pl.pallas_call(
      _logits_body,
      grid=(_B,),
      in_specs=[pl.BlockSpec((1, 512, _DP), lambda b: (b, 0, 0))],
      out_specs=pl.BlockSpec((1, 512, 512), lambda b: (b, 0, 0)),
      out_shape=jax.ShapeDtypeStruct((_B, 512, 512), jnp.float32),
  )(hb)


def kernel(x, edge_index, edge_label_index, atom_tables, conv_W, conv_b,
           bn_gamma, bn_beta, post_W, post_b):
  f32 = jnp.float32
  i32 = jnp.int32

  src = edge_index[0].astype(i32)
  dst = edge_index[1].astype(i32)

  epad = _edge_pad(_E)
  npad = epad - _E
  pad_ar = jnp.arange(npad, dtype=i32)
  src_p = jnp.concatenate([src, (pad_ar * 97) % _N]).reshape(epad // _CHUNK,
                                                            _CHUNK)
  dst_p = jnp.concatenate([dst, _N + pad_ar % (_NP - _N)]).reshape(
      epad // _CHUNK, _CHUNK)

  epad_d = _NC * _NS * _CHUNK * (-(-_E // (_NC * _NS * _CHUNK)))
  pad_ar_d = jnp.arange(epad_d - _E, dtype=i32)
  dst_pd = jnp.concatenate([dst, _N + pad_ar_d % (_NP - _N)]).reshape(
      epad_d // _CHUNK, _CHUNK)

  offs = (jnp.arange(_NFEAT, dtype=i32) * _VOCAB)[:, None]
  src_a = (x.T.astype(i32) + offs).reshape(-1)
  dst_a = jnp.broadcast_to(jnp.arange(_N, dtype=i32)[None],
                           (_NFEAT, _N)).reshape(-1)
  na = _NFEAT * _N
  apad = _edge_pad(na)
  pad_ar_a = jnp.arange(apad - na, dtype=i32)
  src_ap = jnp.concatenate([src_a, pad_ar_a % (_NFEAT * _VOCAB)]).reshape(
      apad // _CHUNK, _CHUNK)
  dst_ap = jnp.concatenate([dst_a, _N + pad_ar_a % (_NP - _N)]).reshape(
      apad // _CHUNK, _CHUNK)

  dpad = _DP - _D
  tflat = jnp.pad(atom_tables.reshape(_NFEAT * _VOCAB, _D), ((0, 0), (0, dpad)))
  t0, t1 = tflat[:, :_HALF], tflat[:, _HALF:]
  wt = jnp.transpose(jnp.pad(conv_W, ((0, 0), (0, dpad), (0, dpad))), (0, 2, 1))
  post_wt = jnp.pad(post_W, ((0, dpad), (0, dpad))).T
  gam = jnp.pad(bn_gamma, ((0, 0), (0, dpad)))
  bet = jnp.pad(bn_beta, ((0, 0), (0, dpad)))
  post_c4 = jnp.pad(post_b, (0, dpad))[None].astype(f32)

  zeros_half = jnp.zeros((_NP, _HALF), f32)
  zeros16 = jnp.zeros((_NP, 16), f32)
  ones_col = jnp.ones((_NP, 1), f32)
  zeros_row = jnp.zeros((1, _DP), f32)

  deg_k = _make_deg(epad_d)
  d0, d1 = deg_k(dst_pd, zeros16)
  dinv = _dinv_call(d0, d1)

  atom_k = _make_gather_scatter(apad)
  cur0, cur1 = atom_k(src_ap, dst_ap, t0, t1, zeros_half, zeros_half)

  agg_k = _make_gather_scatter(epad)
  ones_row = jnp.ones((1, _DP), f32)
  hw0, hw1 = _mlp_call(cur0, cur1, ones_col, dinv, ones_row, zeros_row,
                       wt[0], zeros_row, relu=False)
  for i in range(_NLAYERS):
    a0, a1 = agg_k(src_p, dst_p, hw0, hw1, hw0, hw1)
    last = (i == _NLAYERS - 1)
    hw0, hw1 = _mlpbn_call(
        a0, a1, dinv,
        ones_col if last else dinv,
        gam[i][None], bet[i][None],
        post_wt if last else wt[i + 1],
        post_c4 if last else zeros_row)
  hp0, hp1 = hw0, hw1

  hfull = jnp.concatenate([hp0, hp1], axis=1)[:_N].reshape(_B, _NG, _DP)
  hb = jnp.zeros((_B, 512, _DP), f32).at[:, :_NG, :].set(hfull)
  logits = _logits_call(hb)
  return logits[:, :_NG, :_NG, None]


# speedup vs baseline: 1.0233x; 1.0233x over previous
import functools

import jax
import jax.numpy as jnp
from jax import lax
from jax.experimental import pallas as pl
from jax.experimental.pallas import tpu as pltpu
from jax.experimental.pallas import tpu_sc as plsc

_N = 10000
_E = 160000
_D = 275
_NLAYERS = 5
_NFEAT = 9
_VOCAB = 128
_B = 20
_NG = 500

_NP = 10240
_DP = 288
_HALF = 144
_NC, _NS = 2, 16
_CHUNK = 96
_RPT = _NP // _NS
_BLK = 512
_NB = _NP // _BLK

_mesh = plsc.VectorSubcoreMesh(core_axis_name="c", subcore_axis_name="s",
                               num_cores=_NC, num_subcores=_NS)


def _edge_pad(n_edges):
  unit = _NS * _CHUNK * 4
  return unit * (-(-n_edges // unit))


def _make_gather_scatter(n_edges_pad):
  ept = n_edges_pad // _NS
  cpt = ept // _CHUNK
  kq = cpt // 4

  @functools.partial(
      pl.kernel, mesh=_mesh,
      out_type=(jax.ShapeDtypeStruct((_NP, _HALF), jnp.float32),
                jax.ShapeDtypeStruct((_NP, _HALF), jnp.float32)),
      scratch_types=[
          pltpu.VMEM((4, _CHUNK), jnp.int32),
          pltpu.VMEM((4, _CHUNK), jnp.int32),
          pltpu.VMEM((_CHUNK, _HALF), jnp.float32),
          pltpu.VMEM((_CHUNK, _HALF), jnp.float32),
          pltpu.VMEM_SHARED((_NP, _HALF), jnp.float32),
          (pltpu.SemaphoreType.DMA,) * 4,
          pltpu.SemaphoreType.DMA,
          pltpu.SemaphoreType.DMA,
          pltpu.SemaphoreType.DMA,
          pltpu.SemaphoreType.DMA,
      ],
      compiler_params=pltpu.CompilerParams(use_tc_tiling_on_sc=False))
  def k(src2, dst2, tab0, tab1, init0, init1, out0, out1,
        srcb, dstb, rows0, rows1, acc, semi, semg0, semg1, sems0, sems1):
    cid = lax.axis_index("c")
    sid = lax.axis_index("s")
    r0 = sid * _RPT
    halves = ((tab0, init0, out0), (tab1, init1, out1))
    for h, (tab, init, out) in enumerate(halves):
      @pl.when(cid == h)
      def _(init=init):
        pltpu.sync_copy(init.at[pl.ds(r0, _RPT)], acc.at[pl.ds(r0, _RPT)])
    plsc.subcore_barrier()
    rowbufs = (rows0, rows1)
    semgs = (semg0, semg1)
    semss = (sems0, sems1)
    cbase = sid * cpt

    def load_idx(j, p4):
      pltpu.async_copy(src2.at[cbase + j], srcb.at[p4], semi[p4])
      pltpu.async_copy(dst2.at[cbase + j], dstb.at[p4], semi[p4])

    def wait_idx(p4):
      pltpu.make_async_copy(src2.at[0], srcb.at[p4], semi[p4]).wait()
      pltpu.make_async_copy(dst2.at[0], dstb.at[p4], semi[p4]).wait()

    for h, (tab, init, out) in enumerate(halves):
      @pl.when(cid == h)
      def _(tab=tab):
        load_idx(0, 0)
        load_idx(1, 1)
        wait_idx(0)
        pltpu.async_copy(tab.at[srcb.at[0]], rows0, semg0)
        def outer(kk, carry):
          j0 = 4 * kk
          for r in range(4):
            j = j0 + r
            p2 = r % 2
            p4 = r
            rb, sg, ss = rowbufs[p2], semgs[p2], semss[p2]
            nb, sgn, ssn = rowbufs[1 - p2], semgs[1 - p2], semss[1 - p2]
            @pl.when(j > 0)
            def _():
              pltpu.make_async_copy(nb, acc.at[dstb.at[0]], ssn).wait()
            pltpu.make_async_copy(tab.at[srcb.at[0]], rb, sg).wait()
            pltpu.async_copy(rb, acc.at[dstb.at[p4]], ss, add=True)
            @pl.when(j + 1 < cpt)
            def _():
              wait_idx((p4 + 1) % 4)
              pltpu.async_copy(tab.at[srcb.at[(p4 + 1) % 4]], nb, sgn)
            @pl.when(j + 2 < cpt)
            def _():
              load_idx(j + 2, (p4 + 2) % 4)
          return carry
        lax.fori_loop(0, kq, outer, 0)
        pltpu.make_async_copy(rows1, acc.at[dstb.at[0]], sems1).wait()
    plsc.subcore_barrier()
    for h, (tab, init, out) in enumerate(halves):
      @pl.when(cid == h)
      def _(out=out):
        pltpu.sync_copy(acc.at[pl.ds(r0, _RPT)], out.at[pl.ds(r0, _RPT)])

  return k


def _make_deg(n_edges_pad):
  epw = n_edges_pad // (_NC * _NS)
  cpw = epw // _CHUNK

  @functools.partial(
      pl.kernel, mesh=_mesh,
      out_type=(jax.ShapeDtypeStruct((_NP, 16), jnp.float32),
                jax.ShapeDtypeStruct((_NP, 16), jnp.float32)),
      scratch_types=[
          pltpu.VMEM((cpw, _CHUNK), jnp.int32),
          pltpu.VMEM((_CHUNK, 16), jnp.float32),
          pltpu.VMEM_SHARED((_NP, 16), jnp.float32),
          pltpu.SemaphoreType.DMA,
      ],
      compiler_params=pltpu.CompilerParams(use_tc_tiling_on_sc=False))
  def k(dst2, zeros16, out0, out1, dstb, rows, acc, sem):
    cid = lax.axis_index("c")
    sid = lax.axis_index("s")
    r0 = sid * _RPT
    wid = cid * _NS + sid
    pltpu.sync_copy(dst2.at[pl.ds(wid * cpw, cpw)], dstb)
    pltpu.sync_copy(zeros16.at[pl.ds(r0, _RPT)], acc.at[pl.ds(r0, _RPT)])
    one = jnp.full((16,), 1.0, jnp.float32)
    for r in range(_CHUNK):
      rows[r] = one
    plsc.subcore_barrier()
    def chunk(j, carry):
      @pl.when(j >= 2)
      def _():
        pltpu.make_async_copy(rows, acc.at[dstb.at[0]], sem).wait()
      pltpu.async_copy(rows, acc.at[dstb.at[j]], sem, add=True)
      return carry
    lax.fori_loop(0, cpw, chunk, 0)
    pltpu.make_async_copy(rows, acc.at[dstb.at[0]], sem).wait()
    pltpu.make_async_copy(rows, acc.at[dstb.at[0]], sem).wait()
    plsc.subcore_barrier()
    for h, out in enumerate((out0, out1)):
      @pl.when(cid == h)
      def _(out=out):
        pltpu.sync_copy(acc.at[pl.ds(r0, _RPT)], out.at[pl.ds(r0, _RPT)])

  return k


def _dinv_body(d0_r, d1_r, o_r):
  g = pl.program_id(0)
  deg = 1.0 + d0_r[...][:, :1] + d1_r[...][:, :1]
  row = g * _BLK + lax.broadcasted_iota(jnp.int32, (_BLK, 1), 0)
  o_r[...] = jnp.where(row < _N, lax.rsqrt(deg), 0.0)


def _dinv_call(d0, d1):
  return pl.pallas_call(
      _dinv_body,
      grid=(_NB,),
      in_specs=[pl.BlockSpec((_BLK, 16), lambda g: (g, 0)),
                pl.BlockSpec((_BLK, 16), lambda g: (g, 0))],
      out_specs=pl.BlockSpec((_BLK, 1), lambda g: (g, 0)),
      out_shape=jax.ShapeDtypeStruct((_NP, 1), jnp.float32),
  )(d0, d1)


def _mlp_body(relu, in0_r, in1_r, rsin_r, rsout_r, m_r, a_r, w_r, c4_r,
              o0_r, o1_r):
  t = jnp.concatenate([in0_r[...], in1_r[...]], axis=1)
  t = t * rsin_r[...] * m_r[...] + a_r[...]
  if relu:
    t = jnp.maximum(t, 0.0)
  y = jnp.dot(t, w_r[...], preferred_element_type=jnp.float32)
  y = y * rsout_r[...] + c4_r[...]
  o0_r[...] = y[:, :_HALF]
  o1_r[...] = y[:, _HALF:]


def _mlp_call(in0, in1, rs_in, rs_out, m, a, wt, c4, relu):
  return pl.pallas_call(
      functools.partial(_mlp_body, relu),
      grid=(_NB,),
      in_specs=[
          pl.BlockSpec((_BLK, _HALF), lambda g: (g, 0)),
          pl.BlockSpec((_BLK, _HALF), lambda g: (g, 0)),
          pl.BlockSpec((_BLK, 1), lambda g: (g, 0)),
          pl.BlockSpec((_BLK, 1), lambda g: (g, 0)),
          pl.BlockSpec((1, _DP), lambda g: (0, 0)),
          pl.BlockSpec((1, _DP), lambda g: (0, 0)),
          pl.BlockSpec((_DP, _DP), lambda g: (0, 0)),
          pl.BlockSpec((1, _DP), lambda g: (0, 0)),
      ],
      out_specs=(pl.BlockSpec((_BLK, _HALF), lambda g: (g, 0)),
                 pl.BlockSpec((_BLK, _HALF), lambda g: (g, 0))),
      out_shape=(jax.ShapeDtypeStruct((_NP, _HALF), jnp.float32),
                 jax.ShapeDtypeStruct((_NP, _HALF), jnp.float32)),
  )(in0, in1, rs_in, rs_out, m, a, wt, c4)


def _stats_body(in0_r, in1_r, rs_r, o_r):
  g = pl.program_id(0)
  t = jnp.concatenate([in0_r[...], in1_r[...]], axis=1) * rs_r[...]
  s1 = jnp.sum(t, axis=0, keepdims=True)
  s2 = jnp.sum(t * t, axis=0, keepdims=True)
  @pl.when(g == 0)
  def _():
    o_r[...] = jnp.zeros_like(o_r)
  o_r[0:1, :] = o_r[0:1, :] + s1
  o_r[1:2, :] = o_r[1:2, :] + s2


def _stats_call(in0, in1, rs):
  return pl.pallas_call(
      _stats_body,
      grid=(_NB,),
      in_specs=[pl.BlockSpec((_BLK, _HALF), lambda g: (g, 0)),
                pl.BlockSpec((_BLK, _HALF), lambda g: (g, 0)),
                pl.BlockSpec((_BLK, 1), lambda g: (g, 0))],
      out_specs=pl.BlockSpec((8, _DP), lambda g: (0, 0)),
      out_shape=jax.ShapeDtypeStruct((8, _DP), jnp.float32),
  )(in0, in1, rs)


def _logits_body(h_r, o_r):
  xb = h_r[0]
  o_r[0] = lax.dot_general(xb, xb, (((1,), (1,)), ((), ())),
                           preferred_element_type=jnp.float32)


def _logits_call(hb):
  return pl.pallas_call(
      _logits_body,
      grid=(_B,),
      in_specs=[pl.BlockSpec((1, 512, _DP), lambda b: (b, 0, 0))],
      out_specs=pl.BlockSpec((1, 512, 512), lambda b: (b, 0, 0)),
      out_shape=jax.ShapeDtypeStruct((_B, 512, 512), jnp.float32),
  )(hb)


def kernel(x, edge_index, edge_label_index, atom_tables, conv_W, conv_b,
           bn_gamma, bn_beta, post_W, post_b):
  f32 = jnp.float32
  i32 = jnp.int32

  src = edge_index[0].astype(i32)
  dst = edge_index[1].astype(i32)

  epad = _edge_pad(_E)
  npad = epad - _E
  pad_ar = jnp.arange(npad, dtype=i32)
  src_p = jnp.concatenate([src, (pad_ar * 97) % _N]).reshape(epad // _CHUNK,
                                                            _CHUNK)
  dst_p = jnp.concatenate([dst, _N + pad_ar % (_NP - _N)]).reshape(
      epad // _CHUNK, _CHUNK)

  epad_d = _NC * _NS * _CHUNK * (-(-_E // (_NC * _NS * _CHUNK)))
  pad_ar_d = jnp.arange(epad_d - _E, dtype=i32)
  dst_pd = jnp.concatenate([dst, _N + pad_ar_d % (_NP - _N)]).reshape(
      epad_d // _CHUNK, _CHUNK)

  offs = (jnp.arange(_NFEAT, dtype=i32) * _VOCAB)[:, None]
  src_a = (x.T.astype(i32) + offs).reshape(-1)
  dst_a = jnp.broadcast_to(jnp.arange(_N, dtype=i32)[None],
                           (_NFEAT, _N)).reshape(-1)
  na = _NFEAT * _N
  apad = _edge_pad(na)
  pad_ar_a = jnp.arange(apad - na, dtype=i32)
  src_ap = jnp.concatenate([src_a, pad_ar_a % (_NFEAT * _VOCAB)]).reshape(
      apad // _CHUNK, _CHUNK)
  dst_ap = jnp.concatenate([dst_a, _N + pad_ar_a % (_NP - _N)]).reshape(
      apad // _CHUNK, _CHUNK)

  dpad = _DP - _D
  tflat = jnp.pad(atom_tables.reshape(_NFEAT * _VOCAB, _D), ((0, 0), (0, dpad)))
  t0, t1 = tflat[:, :_HALF], tflat[:, _HALF:]
  wt = jnp.transpose(jnp.pad(conv_W, ((0, 0), (0, dpad), (0, dpad))), (0, 2, 1))
  post_wt = jnp.pad(post_W, ((0, dpad), (0, dpad))).T
  gam = jnp.pad(bn_gamma, ((0, 0), (0, dpad)))
  bet = jnp.pad(bn_beta, ((0, 0), (0, dpad)))
  post_c4 = jnp.pad(post_b, (0, dpad))[None].astype(f32)

  zeros_half = jnp.zeros((_NP, _HALF), f32)
  zeros16 = jnp.zeros((_NP, 16), f32)
  ones_col = jnp.ones((_NP, 1), f32)
  zeros_row = jnp.zeros((1, _DP), f32)

  deg_k = _make_deg(epad_d)
  d0, d1 = deg_k(dst_pd, zeros16)
  dinv = _dinv_call(d0, d1)

  atom_k = _make_gather_scatter(apad)
  cur0, cur1 = atom_k(src_ap, dst_ap, t0, t1, zeros_half, zeros_half)

  agg_k = _make_gather_scatter(epad)
  m = jnp.ones((1, _DP), f32)
  a = jnp.zeros((1, _DP), f32)
  rs_in = ones_col
  for i in range(_NLAYERS):
    hw0, hw1 = _mlp_call(cur0, cur1, rs_in, dinv, m, a, wt[i], zeros_row,
                         relu=(i > 0))
    cur0, cur1 = agg_k(src_p, dst_p, hw0, hw1, hw0, hw1)
    s = _stats_call(cur0, cur1, dinv)
    mu = s[0] / _N
    var = s[1] / _N - mu * mu
    mfac = gam[i] * lax.rsqrt(var + 1e-5)
    m = mfac[None]
    a = (bet[i] - mu * mfac)[None]
    rs_in = dinv

  hp0, hp1 = _mlp_call(cur0, cur1, dinv, ones_col, m, a, post_wt, post_c4,
                       relu=True)

  hfull = jnp.concatenate([hp0, hp1], axis=1)[:_N].reshape(_B, _NG, _DP)
  hb = jnp.zeros((_B, 512, _DP), f32).at[:, :_NG, :].set(hfull)
  logits = _logits_call(hb)
  return logits[:, :_NG, :_NG, None]


# speedup vs baseline: 1.0664x; 1.0422x over previous
import functools

import jax
import jax.numpy as jnp
from jax import lax
from jax.experimental import pallas as pl
from jax.experimental.pallas import tpu as pltpu
from jax.experimental.pallas import tpu_sc as plsc

_N = 10000
_E = 160000
_D = 275
_NLAYERS = 5
_NFEAT = 9
_VOCAB = 128
_B = 20
_NG = 500

_NP = 10240
_DP = 288
_HALF = 144
_NC, _NS = 2, 16
_CHUNK = 96
_RPT = _NP // _NS
_BLK = 1024
_NB = _NP // _BLK

_mesh = plsc.VectorSubcoreMesh(core_axis_name="c", subcore_axis_name="s",
                               num_cores=_NC, num_subcores=_NS)


def _edge_pad(n_edges):
  unit = _NS * _CHUNK * 4
  return unit * (-(-n_edges // unit))


def _make_gather_scatter(n_edges_pad):
  ept = n_edges_pad // _NS
  cpt = ept // _CHUNK
  kq = cpt // 4

  @functools.partial(
      pl.kernel, mesh=_mesh,
      out_type=(jax.ShapeDtypeStruct((_NP, _HALF), jnp.float32),
                jax.ShapeDtypeStruct((_NP, _HALF), jnp.float32)),
      scratch_types=[
          pltpu.VMEM((4, _CHUNK), jnp.int32),
          pltpu.VMEM((4, _CHUNK), jnp.int32),
          pltpu.VMEM((_CHUNK, _HALF), jnp.float32),
          pltpu.VMEM((_CHUNK, _HALF), jnp.float32),
          pltpu.VMEM_SHARED((_NP, _HALF), jnp.float32),
          (pltpu.SemaphoreType.DMA,) * 4,
          pltpu.SemaphoreType.DMA,
          pltpu.SemaphoreType.DMA,
          pltpu.SemaphoreType.DMA,
          pltpu.SemaphoreType.DMA,
      ],
      compiler_params=pltpu.CompilerParams(use_tc_tiling_on_sc=False))
  def k(src2, dst2, tab0, tab1, init0, init1, out0, out1,
        srcb, dstb, rows0, rows1, acc, semi, semg0, semg1, sems0, sems1):
    cid = lax.axis_index("c")
    sid = lax.axis_index("s")
    r0 = sid * _RPT
    halves = ((tab0, init0, out0), (tab1, init1, out1))
    for h, (tab, init, out) in enumerate(halves):
      @pl.when(cid == h)
      def _(init=init):
        pltpu.sync_copy(init.at[pl.ds(r0, _RPT)], acc.at[pl.ds(r0, _RPT)])
    plsc.subcore_barrier()
    rowbufs = (rows0, rows1)
    semgs = (semg0, semg1)
    semss = (sems0, sems1)
    cbase = sid * cpt

    def load_idx(j, p4):
      pltpu.async_copy(src2.at[cbase + j], srcb.at[p4], semi[p4])
      pltpu.async_copy(dst2.at[cbase + j], dstb.at[p4], semi[p4])

    def wait_idx(p4):
      pltpu.make_async_copy(src2.at[0], srcb.at[p4], semi[p4]).wait()
      pltpu.make_async_copy(dst2.at[0], dstb.at[p4], semi[p4]).wait()

    for h, (tab, init, out) in enumerate(halves):
      @pl.when(cid == h)
      def _(tab=tab):
        load_idx(0, 0)
        load_idx(1, 1)
        wait_idx(0)
        pltpu.async_copy(tab.at[srcb.at[0]], rows0, semg0)
        def outer(kk, carry):
          j0 = 4 * kk
          for r in range(4):
            j = j0 + r
            p2 = r % 2
            p4 = r
            rb, sg, ss = rowbufs[p2], semgs[p2], semss[p2]
            nb, sgn, ssn = rowbufs[1 - p2], semgs[1 - p2], semss[1 - p2]
            @pl.when(j > 0)
            def _():
              pltpu.make_async_copy(nb, acc.at[dstb.at[0]], ssn).wait()
            pltpu.make_async_copy(tab.at[srcb.at[0]], rb, sg).wait()
            pltpu.async_copy(rb, acc.at[dstb.at[p4]], ss, add=True)
            @pl.when(j + 1 < cpt)
            def _():
              wait_idx((p4 + 1) % 4)
              pltpu.async_copy(tab.at[srcb.at[(p4 + 1) % 4]], nb, sgn)
            @pl.when(j + 2 < cpt)
            def _():
              load_idx(j + 2, (p4 + 2) % 4)
          return carry
        lax.fori_loop(0, kq, outer, 0)
        pltpu.make_async_copy(rows1, acc.at[dstb.at[0]], sems1).wait()
    plsc.subcore_barrier()
    for h, (tab, init, out) in enumerate(halves):
      @pl.when(cid == h)
      def _(out=out):
        pltpu.sync_copy(acc.at[pl.ds(r0, _RPT)], out.at[pl.ds(r0, _RPT)])

  return k


def _make_deg(n_edges_pad):
  epw = n_edges_pad // (_NC * _NS)
  cpw = epw // _CHUNK

  @functools.partial(
      pl.kernel, mesh=_mesh,
      out_type=(jax.ShapeDtypeStruct((_NP, 16), jnp.float32),
                jax.ShapeDtypeStruct((_NP, 16), jnp.float32)),
      scratch_types=[
          pltpu.VMEM((cpw, _CHUNK), jnp.int32),
          pltpu.VMEM((_CHUNK, 16), jnp.float32),
          pltpu.VMEM_SHARED((_NP, 16), jnp.float32),
          pltpu.SemaphoreType.DMA,
      ],
      compiler_params=pltpu.CompilerParams(use_tc_tiling_on_sc=False))
  def k(dst2, zeros16, out0, out1, dstb, rows, acc, sem):
    cid = lax.axis_index("c")
    sid = lax.axis_index("s")
    r0 = sid * _RPT
    wid = cid * _NS + sid
    pltpu.sync_copy(dst2.at[pl.ds(wid * cpw, cpw)], dstb)
    pltpu.sync_copy(zeros16.at[pl.ds(r0, _RPT)], acc.at[pl.ds(r0, _RPT)])
    one = jnp.full((16,), 1.0, jnp.float32)
    for r in range(_CHUNK):
      rows[r] = one
    plsc.subcore_barrier()
    def chunk(j, carry):
      @pl.when(j >= 2)
      def _():
        pltpu.make_async_copy(rows, acc.at[dstb.at[0]], sem).wait()
      pltpu.async_copy(rows, acc.at[dstb.at[j]], sem, add=True)
      return carry
    lax.fori_loop(0, cpw, chunk, 0)
    pltpu.make_async_copy(rows, acc.at[dstb.at[0]], sem).wait()
    pltpu.make_async_copy(rows, acc.at[dstb.at[0]], sem).wait()
    plsc.subcore_barrier()
    for h, out in enumerate((out0, out1)):
      @pl.when(cid == h)
      def _(out=out):
        pltpu.sync_copy(acc.at[pl.ds(r0, _RPT)], out.at[pl.ds(r0, _RPT)])

  return k


def _dinv_body(d0_r, d1_r, o_r):
  g = pl.program_id(0)
  deg = 1.0 + d0_r[...][:, :1] + d1_r[...][:, :1]
  row = g * _BLK + lax.broadcasted_iota(jnp.int32, (_BLK, 1), 0)
  o_r[...] = jnp.where(row < _N, lax.rsqrt(deg), 0.0)


def _dinv_call(d0, d1):
  return pl.pallas_call(
      _dinv_body,
      grid=(_NB,),
      in_specs=[pl.BlockSpec((_BLK, 16), lambda g: (g, 0)),
                pl.BlockSpec((_BLK, 16), lambda g: (g, 0))],
      out_specs=pl.BlockSpec((_BLK, 1), lambda g: (g, 0)),
      out_shape=jax.ShapeDtypeStruct((_NP, 1), jnp.float32),
  )(d0, d1)


def _mlp_body(relu, in0_r, in1_r, rsin_r, rsout_r, m_r, a_r, w_r, c4_r,
              o0_r, o1_r):
  t = jnp.concatenate([in0_r[...], in1_r[...]], axis=1)
  t = t * rsin_r[...] * m_r[...] + a_r[...]
  if relu:
    t = jnp.maximum(t, 0.0)
  y = jnp.dot(t, w_r[...], preferred_element_type=jnp.float32)
  y = y * rsout_r[...] + c4_r[...]
  o0_r[...] = y[:, :_HALF]
  o1_r[...] = y[:, _HALF:]


def _mlp_call(in0, in1, rs_in, rs_out, m, a, wt, c4, relu):
  return pl.pallas_call(
      functools.partial(_mlp_body, relu),
      grid=(_NB,),
      in_specs=[
          pl.BlockSpec((_BLK, _HALF), lambda g: (g, 0)),
          pl.BlockSpec((_BLK, _HALF), lambda g: (g, 0)),
          pl.BlockSpec((_BLK, 1), lambda g: (g, 0)),
          pl.BlockSpec((_BLK, 1), lambda g: (g, 0)),
          pl.BlockSpec((1, _DP), lambda g: (0, 0)),
          pl.BlockSpec((1, _DP), lambda g: (0, 0)),
          pl.BlockSpec((_DP, _DP), lambda g: (0, 0)),
          pl.BlockSpec((1, _DP), lambda g: (0, 0)),
      ],
      out_specs=(pl.BlockSpec((_BLK, _HALF), lambda g: (g, 0)),
                 pl.BlockSpec((_BLK, _HALF), lambda g: (g, 0))),
      out_shape=(jax.ShapeDtypeStruct((_NP, _HALF), jnp.float32),
                 jax.ShapeDtypeStruct((_NP, _HALF), jnp.float32)),
  )(in0, in1, rs_in, rs_out, m, a, wt, c4)


def _stats_body(in0_r, in1_r, rs_r, o_r):
  g = pl.program_id(0)
  t = jnp.concatenate([in0_r[...], in1_r[...]], axis=1) * rs_r[...]
  s1 = jnp.sum(t, axis=0, keepdims=True)
  s2 = jnp.sum(t * t, axis=0, keepdims=True)
  @pl.when(g == 0)
  def _():
    o_r[...] = jnp.zeros_like(o_r)
  o_r[0:1, :] = o_r[0:1, :] + s1
  o_r[1:2, :] = o_r[1:2, :] + s2


def _stats_call(in0, in1, rs):
  return pl.pallas_call(
      _stats_body,
      grid=(_NB,),
      in_specs=[pl.BlockSpec((_BLK, _HALF), lambda g: (g, 0)),
                pl.BlockSpec((_BLK, _HALF), lambda g: (g, 0)),
                pl.BlockSpec((_BLK, 1), lambda g: (g, 0))],
      out_specs=pl.BlockSpec((8, _DP), lambda g: (0, 0)),
      out_shape=jax.ShapeDtypeStruct((8, _DP), jnp.float32),
  )(in0, in1, rs)


def _logits_body(h_r, o_r):
  xb = h_r[0]
  o_r[0] = lax.dot_general(xb, xb, (((1,), (1,)), ((), ())),
                           preferred_element_type=jnp.float32)


def _logits_call(hb):
  return pl.pallas_call(
      _logits_body,
      grid=(_B,),
      in_specs=[pl.BlockSpec((1, 512, _DP), lambda b: (b, 0, 0))],
      out_specs=pl.BlockSpec((1, 512, 512), lambda b: (b, 0, 0)),
      out_shape=jax.ShapeDtypeStruct((_B, 512, 512), jnp.float32),
  )(hb)


def kernel(x, edge_index, edge_label_index, atom_tables, conv_W, conv_b,
           bn_gamma, bn_beta, post_W, post_b):
  f32 = jnp.float32
  i32 = jnp.int32

  src = edge_index[0].astype(i32)
  dst = edge_index[1].astype(i32)

  epad = _edge_pad(_E)
  npad = epad - _E
  pad_ar = jnp.arange(npad, dtype=i32)
  src_p = jnp.concatenate([src, (pad_ar * 97) % _N]).reshape(epad // _CHUNK,
                                                            _CHUNK)
  dst_p = jnp.concatenate([dst, _N + pad_ar % (_NP - _N)]).reshape(
      epad // _CHUNK, _CHUNK)

  epad_d = _NC * _NS * _CHUNK * (-(-_E // (_NC * _NS * _CHUNK)))
  pad_ar_d = jnp.arange(epad_d - _E, dtype=i32)
  dst_pd = jnp.concatenate([dst, _N + pad_ar_d % (_NP - _N)]).reshape(
      epad_d // _CHUNK, _CHUNK)

  offs = (jnp.arange(_NFEAT, dtype=i32) * _VOCAB)[:, None]
  src_a = (x.T.astype(i32) + offs).reshape(-1)
  dst_a = jnp.broadcast_to(jnp.arange(_N, dtype=i32)[None],
                           (_NFEAT, _N)).reshape(-1)
  na = _NFEAT * _N
  apad = _edge_pad(na)
  pad_ar_a = jnp.arange(apad - na, dtype=i32)
  src_ap = jnp.concatenate([src_a, pad_ar_a % (_NFEAT * _VOCAB)]).reshape(
      apad // _CHUNK, _CHUNK)
  dst_ap = jnp.concatenate([dst_a, _N + pad_ar_a % (_NP - _N)]).reshape(
      apad // _CHUNK, _CHUNK)

  dpad = _DP - _D
  tflat = jnp.pad(atom_tables.reshape(_NFEAT * _VOCAB, _D), ((0, 0), (0, dpad)))
  t0, t1 = tflat[:, :_HALF], tflat[:, _HALF:]
  wt = jnp.transpose(jnp.pad(conv_W, ((0, 0), (0, dpad), (0, dpad))), (0, 2, 1))
  post_wt = jnp.pad(post_W, ((0, dpad), (0, dpad))).T
  gam = jnp.pad(bn_gamma, ((0, 0), (0, dpad)))
  bet = jnp.pad(bn_beta, ((0, 0), (0, dpad)))
  post_c4 = jnp.pad(post_b, (0, dpad))[None].astype(f32)

  zeros_half = jnp.zeros((_NP, _HALF), f32)
  zeros16 = jnp.zeros((_NP, 16), f32)
  ones_col = jnp.ones((_NP, 1), f32)
  zeros_row = jnp.zeros((1, _DP), f32)

  deg_k = _make_deg(epad_d)
  d0, d1 = deg_k(dst_pd, zeros16)
  dinv = _dinv_call(d0, d1)

  atom_k = _make_gather_scatter(apad)
  cur0, cur1 = atom_k(src_ap, dst_ap, t0, t1, zeros_half, zeros_half)

  agg_k = _make_gather_scatter(epad)
  m = jnp.ones((1, _DP), f32)
  a = jnp.zeros((1, _DP), f32)
  rs_in = ones_col
  for i in range(_NLAYERS):
    hw0, hw1 = _mlp_call(cur0, cur1, rs_in, dinv, m, a, wt[i], zeros_row,
                         relu=(i > 0))
    cur0, cur1 = agg_k(src_p, dst_p, hw0, hw1, hw0, hw1)
    s = _stats_call(cur0, cur1, dinv)
    mu = s[0] / _N
    var = s[1] / _N - mu * mu
    mfac = gam[i] * lax.rsqrt(var + 1e-5)
    m = mfac[None]
    a = (bet[i] - mu * mfac)[None]
    rs_in = dinv

  hp0, hp1 = _mlp_call(cur0, cur1, dinv, ones_col, m, a, post_wt, post_c4,
                       relu=True)

  hfull = jnp.concatenate([hp0, hp1], axis=1)[:_N].reshape(_B, _NG, _DP)
  hb = jnp.zeros((_B, 512, _DP), f32).at[:, :_NG, :].set(hfull)
  logits = _logits_call(hb)
  return logits[:, :_NG, :_NG, None]


# speedup vs baseline: 1.0802x; 1.0129x over previous
import functools

import jax
import jax.numpy as jnp
from jax import lax
from jax.experimental import pallas as pl
from jax.experimental.pallas import tpu as pltpu
from jax.experimental.pallas import tpu_sc as plsc

_N = 10000
_E = 160000
_D = 275
_NLAYERS = 5
_NFEAT = 9
_VOCAB = 128
_B = 20
_NG = 500

_NP = 10240
_DP = 288
_HALF = 144
_NC, _NS = 2, 16
_CHUNK = 96
_RPT = _NP // _NS
_BLK = 2048
_NB = _NP // _BLK

_mesh = plsc.VectorSubcoreMesh(core_axis_name="c", subcore_axis_name="s",
                               num_cores=_NC, num_subcores=_NS)


def _edge_pad(n_edges):
  unit = _NS * _CHUNK * 4
  return unit * (-(-n_edges // unit))


def _make_gather_scatter(n_edges_pad):
  ept = n_edges_pad // _NS
  cpt = ept // _CHUNK
  kq = cpt // 4

  @functools.partial(
      pl.kernel, mesh=_mesh,
      out_type=(jax.ShapeDtypeStruct((_NP, _HALF), jnp.float32),
                jax.ShapeDtypeStruct((_NP, _HALF), jnp.float32)),
      scratch_types=[
          pltpu.VMEM((4, _CHUNK), jnp.int32),
          pltpu.VMEM((4, _CHUNK), jnp.int32),
          pltpu.VMEM((_CHUNK, _HALF), jnp.float32),
          pltpu.VMEM((_CHUNK, _HALF), jnp.float32),
          pltpu.VMEM_SHARED((_NP, _HALF), jnp.float32),
          (pltpu.SemaphoreType.DMA,) * 4,
          pltpu.SemaphoreType.DMA,
          pltpu.SemaphoreType.DMA,
          pltpu.SemaphoreType.DMA,
          pltpu.SemaphoreType.DMA,
      ],
      compiler_params=pltpu.CompilerParams(use_tc_tiling_on_sc=False))
  def k(src2, dst2, tab0, tab1, init0, init1, out0, out1,
        srcb, dstb, rows0, rows1, acc, semi, semg0, semg1, sems0, sems1):
    cid = lax.axis_index("c")
    sid = lax.axis_index("s")
    r0 = sid * _RPT
    halves = ((tab0, init0, out0), (tab1, init1, out1))
    for h, (tab, init, out) in enumerate(halves):
      @pl.when(cid == h)
      def _(init=init):
        pltpu.sync_copy(init.at[pl.ds(r0, _RPT)], acc.at[pl.ds(r0, _RPT)])
    plsc.subcore_barrier()
    rowbufs = (rows0, rows1)
    semgs = (semg0, semg1)
    semss = (sems0, sems1)
    cbase = sid * cpt

    def load_idx(j, p4):
      pltpu.async_copy(src2.at[cbase + j], srcb.at[p4], semi[p4])
      pltpu.async_copy(dst2.at[cbase + j], dstb.at[p4], semi[p4])

    def wait_idx(p4):
      pltpu.make_async_copy(src2.at[0], srcb.at[p4], semi[p4]).wait()
      pltpu.make_async_copy(dst2.at[0], dstb.at[p4], semi[p4]).wait()

    for h, (tab, init, out) in enumerate(halves):
      @pl.when(cid == h)
      def _(tab=tab):
        load_idx(0, 0)
        load_idx(1, 1)
        wait_idx(0)
        pltpu.async_copy(tab.at[srcb.at[0]], rows0, semg0)
        def outer(kk, carry):
          j0 = 4 * kk
          for r in range(4):
            j = j0 + r
            p2 = r % 2
            p4 = r
            rb, sg, ss = rowbufs[p2], semgs[p2], semss[p2]
            nb, sgn, ssn = rowbufs[1 - p2], semgs[1 - p2], semss[1 - p2]
            @pl.when(j > 0)
            def _():
              pltpu.make_async_copy(nb, acc.at[dstb.at[0]], ssn).wait()
            pltpu.make_async_copy(tab.at[srcb.at[0]], rb, sg).wait()
            pltpu.async_copy(rb, acc.at[dstb.at[p4]], ss, add=True)
            @pl.when(j + 1 < cpt)
            def _():
              wait_idx((p4 + 1) % 4)
              pltpu.async_copy(tab.at[srcb.at[(p4 + 1) % 4]], nb, sgn)
            @pl.when(j + 2 < cpt)
            def _():
              load_idx(j + 2, (p4 + 2) % 4)
          return carry
        lax.fori_loop(0, kq, outer, 0)
        pltpu.make_async_copy(rows1, acc.at[dstb.at[0]], sems1).wait()
    plsc.subcore_barrier()
    for h, (tab, init, out) in enumerate(halves):
      @pl.when(cid == h)
      def _(out=out):
        pltpu.sync_copy(acc.at[pl.ds(r0, _RPT)], out.at[pl.ds(r0, _RPT)])

  return k


def _make_deg(n_edges_pad):
  epw = n_edges_pad // (_NC * _NS)
  cpw = epw // _CHUNK

  @functools.partial(
      pl.kernel, mesh=_mesh,
      out_type=(jax.ShapeDtypeStruct((_NP, 16), jnp.float32),
                jax.ShapeDtypeStruct((_NP, 16), jnp.float32)),
      scratch_types=[
          pltpu.VMEM((cpw, _CHUNK), jnp.int32),
          pltpu.VMEM((_CHUNK, 16), jnp.float32),
          pltpu.VMEM_SHARED((_NP, 16), jnp.float32),
          pltpu.SemaphoreType.DMA,
      ],
      compiler_params=pltpu.CompilerParams(use_tc_tiling_on_sc=False))
  def k(dst2, zeros16, out0, out1, dstb, rows, acc, sem):
    cid = lax.axis_index("c")
    sid = lax.axis_index("s")
    r0 = sid * _RPT
    wid = cid * _NS + sid
    pltpu.sync_copy(dst2.at[pl.ds(wid * cpw, cpw)], dstb)
    pltpu.sync_copy(zeros16.at[pl.ds(r0, _RPT)], acc.at[pl.ds(r0, _RPT)])
    one = jnp.full((16,), 1.0, jnp.float32)
    for r in range(_CHUNK):
      rows[r] = one
    plsc.subcore_barrier()
    def chunk(j, carry):
      @pl.when(j >= 2)
      def _():
        pltpu.make_async_copy(rows, acc.at[dstb.at[0]], sem).wait()
      pltpu.async_copy(rows, acc.at[dstb.at[j]], sem, add=True)
      return carry
    lax.fori_loop(0, cpw, chunk, 0)
    pltpu.make_async_copy(rows, acc.at[dstb.at[0]], sem).wait()
    pltpu.make_async_copy(rows, acc.at[dstb.at[0]], sem).wait()
    plsc.subcore_barrier()
    for h, out in enumerate((out0, out1)):
      @pl.when(cid == h)
      def _(out=out):
        pltpu.sync_copy(acc.at[pl.ds(r0, _RPT)], out.at[pl.ds(r0, _RPT)])

  return k


def _dinv_body(d0_r, d1_r, o_r):
  g = pl.program_id(0)
  deg = 1.0 + d0_r[...][:, :1] + d1_r[...][:, :1]
  row = g * _BLK + lax.broadcasted_iota(jnp.int32, (_BLK, 1), 0)
  o_r[...] = jnp.where(row < _N, lax.rsqrt(deg), 0.0)


def _dinv_call(d0, d1):
  return pl.pallas_call(
      _dinv_body,
      grid=(_NB,),
      in_specs=[pl.BlockSpec((_BLK, 16), lambda g: (g, 0)),
                pl.BlockSpec((_BLK, 16), lambda g: (g, 0))],
      out_specs=pl.BlockSpec((_BLK, 1), lambda g: (g, 0)),
      out_shape=jax.ShapeDtypeStruct((_NP, 1), jnp.float32),
  )(d0, d1)


def _mlp_body(relu, in0_r, in1_r, rsin_r, rsout_r, m_r, a_r, w_r, c4_r,
              o0_r, o1_r):
  t = jnp.concatenate([in0_r[...], in1_r[...]], axis=1)
  t = t * rsin_r[...] * m_r[...] + a_r[...]
  if relu:
    t = jnp.maximum(t, 0.0)
  y = jnp.dot(t, w_r[...], preferred_element_type=jnp.float32)
  y = y * rsout_r[...] + c4_r[...]
  o0_r[...] = y[:, :_HALF]
  o1_r[...] = y[:, _HALF:]


def _mlp_call(in0, in1, rs_in, rs_out, m, a, wt, c4, relu):
  return pl.pallas_call(
      functools.partial(_mlp_body, relu),
      grid=(_NB,),
      in_specs=[
          pl.BlockSpec((_BLK, _HALF), lambda g: (g, 0)),
          pl.BlockSpec((_BLK, _HALF), lambda g: (g, 0)),
          pl.BlockSpec((_BLK, 1), lambda g: (g, 0)),
          pl.BlockSpec((_BLK, 1), lambda g: (g, 0)),
          pl.BlockSpec((1, _DP), lambda g: (0, 0)),
          pl.BlockSpec((1, _DP), lambda g: (0, 0)),
          pl.BlockSpec((_DP, _DP), lambda g: (0, 0)),
          pl.BlockSpec((1, _DP), lambda g: (0, 0)),
      ],
      out_specs=(pl.BlockSpec((_BLK, _HALF), lambda g: (g, 0)),
                 pl.BlockSpec((_BLK, _HALF), lambda g: (g, 0))),
      out_shape=(jax.ShapeDtypeStruct((_NP, _HALF), jnp.float32),
                 jax.ShapeDtypeStruct((_NP, _HALF), jnp.float32)),
  )(in0, in1, rs_in, rs_out, m, a, wt, c4)


def _stats_body(in0_r, in1_r, rs_r, o_r):
  g = pl.program_id(0)
  t = jnp.concatenate([in0_r[...], in1_r[...]], axis=1) * rs_r[...]
  s1 = jnp.sum(t, axis=0, keepdims=True)
  s2 = jnp.sum(t * t, axis=0, keepdims=True)
  @pl.when(g == 0)
  def _():
    o_r[...] = jnp.zeros_like(o_r)
  o_r[0:1, :] = o_r[0:1, :] + s1
  o_r[1:2, :] = o_r[1:2, :] + s2


def _stats_call(in0, in1, rs):
  return pl.pallas_call(
      _stats_body,
      grid=(_NB,),
      in_specs=[pl.BlockSpec((_BLK, _HALF), lambda g: (g, 0)),
                pl.BlockSpec((_BLK, _HALF), lambda g: (g, 0)),
                pl.BlockSpec((_BLK, 1), lambda g: (g, 0))],
      out_specs=pl.BlockSpec((8, _DP), lambda g: (0, 0)),
      out_shape=jax.ShapeDtypeStruct((8, _DP), jnp.float32),
  )(in0, in1, rs)


def _logits_body(h_r, o_r):
  xb = h_r[0]
  o_r[0] = lax.dot_general(xb, xb, (((1,), (1,)), ((), ())),
                           preferred_element_type=jnp.float32)


def _logits_call(hb):
  return pl.pallas_call(
      _logits_body,
      grid=(_B,),
      in_specs=[pl.BlockSpec((1, 512, _DP), lambda b: (b, 0, 0))],
      out_specs=pl.BlockSpec((1, 512, 512), lambda b: (b, 0, 0)),
      out_shape=jax.ShapeDtypeStruct((_B, 512, 512), jnp.float32),
  )(hb)


def kernel(x, edge_index, edge_label_index, atom_tables, conv_W, conv_b,
           bn_gamma, bn_beta, post_W, post_b):
  f32 = jnp.float32
  i32 = jnp.int32

  src = edge_index[0].astype(i32)
  dst = edge_index[1].astype(i32)

  epad = _edge_pad(_E)
  npad = epad - _E
  pad_ar = jnp.arange(npad, dtype=i32)
  src_p = jnp.concatenate([src, (pad_ar * 97) % _N]).reshape(epad // _CHUNK,
                                                            _CHUNK)
  dst_p = jnp.concatenate([dst, _N + pad_ar % (_NP - _N)]).reshape(
      epad // _CHUNK, _CHUNK)

  epad_d = _NC * _NS * _CHUNK * (-(-_E // (_NC * _NS * _CHUNK)))
  pad_ar_d = jnp.arange(epad_d - _E, dtype=i32)
  dst_pd = jnp.concatenate([dst, _N + pad_ar_d % (_NP - _N)]).reshape(
      epad_d // _CHUNK, _CHUNK)

  offs = (jnp.arange(_NFEAT, dtype=i32) * _VOCAB)[:, None]
  src_a = (x.T.astype(i32) + offs).reshape(-1)
  dst_a = jnp.broadcast_to(jnp.arange(_N, dtype=i32)[None],
                           (_NFEAT, _N)).reshape(-1)
  na = _NFEAT * _N
  apad = _edge_pad(na)
  pad_ar_a = jnp.arange(apad - na, dtype=i32)
  src_ap = jnp.concatenate([src_a, pad_ar_a % (_NFEAT * _VOCAB)]).reshape(
      apad // _CHUNK, _CHUNK)
  dst_ap = jnp.concatenate([dst_a, _N + pad_ar_a % (_NP - _N)]).reshape(
      apad // _CHUNK, _CHUNK)

  dpad = _DP - _D
  tflat = jnp.pad(atom_tables.reshape(_NFEAT * _VOCAB, _D), ((0, 0), (0, dpad)))
  t0, t1 = tflat[:, :_HALF], tflat[:, _HALF:]
  wt = jnp.transpose(jnp.pad(conv_W, ((0, 0), (0, dpad), (0, dpad))), (0, 2, 1))
  post_wt = jnp.pad(post_W, ((0, dpad), (0, dpad))).T
  gam = jnp.pad(bn_gamma, ((0, 0), (0, dpad)))
  bet = jnp.pad(bn_beta, ((0, 0), (0, dpad)))
  post_c4 = jnp.pad(post_b, (0, dpad))[None].astype(f32)

  zeros_half = jnp.zeros((_NP, _HALF), f32)
  zeros16 = jnp.zeros((_NP, 16), f32)
  ones_col = jnp.ones((_NP, 1), f32)
  zeros_row = jnp.zeros((1, _DP), f32)

  deg_k = _make_deg(epad_d)
  d0, d1 = deg_k(dst_pd, zeros16)
  dinv = _dinv_call(d0, d1)

  atom_k = _make_gather_scatter(apad)
  cur0, cur1 = atom_k(src_ap, dst_ap, t0, t1, zeros_half, zeros_half)

  agg_k = _make_gather_scatter(epad)
  m = jnp.ones((1, _DP), f32)
  a = jnp.zeros((1, _DP), f32)
  rs_in = ones_col
  for i in range(_NLAYERS):
    hw0, hw1 = _mlp_call(cur0, cur1, rs_in, dinv, m, a, wt[i], zeros_row,
                         relu=(i > 0))
    cur0, cur1 = agg_k(src_p, dst_p, hw0, hw1, hw0, hw1)
    s = _stats_call(cur0, cur1, dinv)
    mu = s[0] / _N
    var = s[1] / _N - mu * mu
    mfac = gam[i] * lax.rsqrt(var + 1e-5)
    m = mfac[None]
    a = (bet[i] - mu * mfac)[None]
    rs_in = dinv

  hp0, hp1 = _mlp_call(cur0, cur1, dinv, ones_col, m, a, post_wt, post_c4,
                       relu=True)

  hfull = jnp.concatenate([hp0, hp1], axis=1)[:_N].reshape(_B, _NG, _DP)
  hb = jnp.zeros((_B, 512, _DP), f32).at[:, :_NG, :].set(hfull)
  logits = _logits_call(hb)
  return logits[:, :_NG, :_NG, None]
